# baseline (device time: 625027 ns/iter reference)
import jax
import jax.numpy as jnp
from jax import lax
from jax.experimental import pallas as pl
from jax.experimental.pallas import tpu as pltpu

N_DEV = 8
M_TILE = 256
H_TILE = 512


def _mlp_body(x_ref, wg_ref, wu_ref, wd_ref, out_ref):
    j = pl.program_id(1)
    x = x_ref[...]
    gate = jnp.dot(x, wg_ref[...], preferred_element_type=jnp.float32)
    up = jnp.dot(x, wu_ref[...], preferred_element_type=jnp.float32)
    h = gate * (up * jax.nn.sigmoid(up))
    acc = jnp.dot(h, wd_ref[...], preferred_element_type=jnp.float32)

    @pl.when(j == 0)
    def _():
        out_ref[...] = acc

    @pl.when(j != 0)
    def _():
        out_ref[...] += acc


def _partial_mlp(x, Wg, Wu, Wd):
    m, k = x.shape
    _, h = Wg.shape
    _, d = Wd.shape
    grid = (m // M_TILE, h // H_TILE)
    return pl.pallas_call(
        _mlp_body,
        grid=grid,
        in_specs=[
            pl.BlockSpec((M_TILE, k), lambda mi, j: (mi, 0)),
            pl.BlockSpec((k, H_TILE), lambda mi, j: (0, j)),
            pl.BlockSpec((k, H_TILE), lambda mi, j: (0, j)),
            pl.BlockSpec((H_TILE, d), lambda mi, j: (j, 0)),
        ],
        out_specs=pl.BlockSpec((M_TILE, d), lambda mi, j: (mi, 0)),
        out_shape=jax.ShapeDtypeStruct((m, d), jnp.float32),
        compiler_params=pltpu.CompilerParams(
            dimension_semantics=("arbitrary", "arbitrary"),
        ),
    )(x, Wg, Wu, Wd)


def _allreduce_body(in_ref, out_ref, recv_ref,
                    rs_send, rs_recv, ag_send, ag_recv):
    me = lax.axis_index("i")
    left = lax.rem(me + N_DEV - 1, N_DEV)
    right = lax.rem(me + 1, N_DEV)
    rows = in_ref.shape[0] // N_DEV

    barrier = pltpu.get_barrier_semaphore()
    for nbr in (left, right):
        pl.semaphore_signal(
            barrier, inc=1,
            device_id=(nbr,), device_id_type=pl.DeviceIdType.MESH,
        )
    pl.semaphore_wait(barrier, 2)

    out_ref[...] = in_ref[...]

    for s in range(N_DEV - 1):
        send_idx = lax.rem(me - s + N_DEV, N_DEV)
        rdma = pltpu.make_async_remote_copy(
            src_ref=out_ref.at[pl.ds(send_idx * rows, rows), :],
            dst_ref=recv_ref.at[s],
            send_sem=rs_send.at[s],
            recv_sem=rs_recv.at[s],
            device_id=(right,),
            device_id_type=pl.DeviceIdType.MESH,
        )
        rdma.start()
        rdma.wait()
        recv_idx = lax.rem(me - s - 1 + N_DEV, N_DEV)
        out_ref[pl.ds(recv_idx * rows, rows), :] += recv_ref[s]

    for t in range(N_DEV - 1):
        g = lax.rem(me + 1 - t + N_DEV, N_DEV)
        rdma = pltpu.make_async_remote_copy(
            src_ref=out_ref.at[pl.ds(g * rows, rows), :],
            dst_ref=out_ref.at[pl.ds(g * rows, rows), :],
            send_sem=ag_send.at[t],
            recv_sem=ag_recv.at[t],
            device_id=(right,),
            device_id_type=pl.DeviceIdType.MESH,
        )
        rdma.start()
        rdma.wait()


def _allreduce(partial):
    m, d = partial.shape
    rows = m // N_DEV
    return pl.pallas_call(
        _allreduce_body,
        out_shape=jax.ShapeDtypeStruct((m, d), jnp.float32),
        in_specs=[pl.BlockSpec(memory_space=pltpu.VMEM)],
        out_specs=pl.BlockSpec(memory_space=pltpu.VMEM),
        scratch_shapes=[
            pltpu.VMEM((N_DEV - 1, rows, d), jnp.float32),
            pltpu.SemaphoreType.DMA((N_DEV - 1,)),
            pltpu.SemaphoreType.DMA((N_DEV - 1,)),
            pltpu.SemaphoreType.DMA((N_DEV - 1,)),
            pltpu.SemaphoreType.DMA((N_DEV - 1,)),
        ],
        compiler_params=pltpu.CompilerParams(collective_id=0),
    )(partial)


def kernel(x, Wg, Wu, Wd):
    partial = _partial_mlp(x, Wg, Wu, Wd)
    return _allreduce(partial)
